# baseline (speedup 1.0000x reference)
import functools

import jax
import jax.numpy as jnp
from jax import lax
from jax.experimental import pallas as pl
from jax.experimental.pallas import tpu as pltpu
from jax.experimental.pallas import tpu_sc as plsc

N = 10000
NPAD = 10112
F_IN = 128
H = 64
C = 40
E = 320000
EP = 100000

NC = 2
NS = 16
NT = NC * NS
RPT = NPAD // NS

CHUNK = 128
NBUF = 4
LBUF = 2

NCH_E = 2560
EPAD = NCH_E * CHUNK
Q0_E = 152
Q1_E = NCH_E // NS - Q0_E
QM_E = max(Q0_E, Q1_E)

NCH_L = 1600
LPAD = NCH_L * CHUNK
Q0_L = 92
Q1_L = NCH_L // NS - Q0_L
QM_L = max(Q0_L, Q1_L)

_mesh = plsc.VectorSubcoreMesh(
    core_axis_name="c", subcore_axis_name="s", num_cores=NC, num_subcores=NS)


def _tile_ids():
    c = lax.axis_index("c")
    s = lax.axis_index("s")
    return c, s, c * NS + s


def _split(c, s, q0, q1):
    q = jnp.where(c == 0, q0, q1)
    start = jnp.where(c == 0, s * q0, NS * q0 + s * q1)
    return q, start


def _deg_body(dsts_hbm, out_hbm, didx, deg):
    c, s, t = _tile_ids()
    cpt = NCH_E // NT
    pltpu.sync_copy(dsts_hbm.at[pl.ds(t * cpt, cpt)], didx)

    def z(i, carry):
        deg[pl.ds(i * 16, 16)] = jnp.zeros((16,), jnp.float32)
        return carry

    lax.fori_loop(0, NPAD // 16, z, 0)

    def g(i, carry):
        iv = didx[i // (CHUNK // 16), pl.ds((i % (CHUNK // 16)) * 16, 16)]
        plsc.addupdate_scatter(deg, [iv], jnp.ones((16,), jnp.float32))
        return carry

    lax.fori_loop(0, cpt * (CHUNK // 16), g, 0)
    pltpu.sync_copy(deg, out_hbm.at[t])


_deg_call = pl.kernel(
    _deg_body,
    out_type=jax.ShapeDtypeStruct((NT, NPAD), jnp.float32),
    mesh=_mesh,
    compiler_params=pltpu.CompilerParams(
        use_tc_tiling_on_sc=False, needs_layout_passes=False),
    scratch_types=[
        pltpu.VMEM((NCH_E // NT, CHUNK), jnp.int32),
        pltpu.VMEM((NPAD,), jnp.float32),
    ],
)


def _prop_body(srcs_hbm, dsts_hbm, y_hbm, zeros_hbm, out_hbm,
               sidx, didx, r0, r1, r2, r3, acc, g0, g1, g2, g3):
    c, s, t = _tile_ids()
    rows = [r0, r1, r2, r3]
    gsem = [g0, g1, g2, g3]
    q, start = _split(c, s, Q0_E, Q1_E)
    pltpu.sync_copy(srcs_hbm.at[pl.ds(start, QM_E)], sidx)
    pltpu.sync_copy(dsts_hbm.at[pl.ds(start, QM_E)], didx)
    pltpu.sync_copy(zeros_hbm.at[pl.ds(s * RPT, RPT)], acc.at[pl.ds(s * RPT, RPT)])
    plsc.subcore_barrier()
    for b in range(NBUF):
        pltpu.make_async_copy(y_hbm.at[sidx.at[b]], rows[b], gsem[b]).start()

    def grp(g, carry):
        for b in range(NBUF):
            k = g * NBUF + b
            pltpu.make_async_copy(y_hbm.at[sidx.at[k]], rows[b], gsem[b]).wait()
            pltpu.sync_copy(rows[b], acc.at[didx.at[k]], add=True)

            @pl.when(k + NBUF < q)
            def _():
                pltpu.make_async_copy(
                    y_hbm.at[sidx.at[k + NBUF]], rows[b], gsem[b]).start()
        return carry

    lax.fori_loop(0, q // NBUF, grp, 0)
    plsc.subcore_barrier()
    pltpu.sync_copy(acc.at[pl.ds(s * RPT, RPT)], out_hbm.at[c, pl.ds(s * RPT, RPT)])


_prop_call = pl.kernel(
    _prop_body,
    out_type=jax.ShapeDtypeStruct((NC, NPAD, H), jnp.float32),
    mesh=_mesh,
    compiler_params=pltpu.CompilerParams(use_tc_tiling_on_sc=False),
    scratch_types=[
        pltpu.VMEM((QM_E, CHUNK), jnp.int32),
        pltpu.VMEM((QM_E, CHUNK), jnp.int32),
        pltpu.VMEM((CHUNK, H), jnp.float32),
        pltpu.VMEM((CHUNK, H), jnp.float32),
        pltpu.VMEM((CHUNK, H), jnp.float32),
        pltpu.VMEM((CHUNK, H), jnp.float32),
        pltpu.VMEM_SHARED((NPAD, H), jnp.float32),
        pltpu.SemaphoreType.DMA,
        pltpu.SemaphoreType.DMA,
        pltpu.SemaphoreType.DMA,
        pltpu.SemaphoreType.DMA,
    ],
)


def _link_body(li_hbm, lj_hbm, feat_hbm, out_hbm, ai, aj,
               ra0, ra1, rb0, rb1, sbuf, ga0, ga1, gb0, gb1):
    c, s, t = _tile_ids()
    rowsa = [ra0, ra1]
    rowsb = [rb0, rb1]
    gsa = [ga0, ga1]
    gsb = [gb0, gb1]
    q, start = _split(c, s, Q0_L, Q1_L)
    pltpu.sync_copy(li_hbm.at[pl.ds(start, QM_L)], ai)
    pltpu.sync_copy(lj_hbm.at[pl.ds(start, QM_L)], aj)
    for b in range(LBUF):
        pltpu.make_async_copy(feat_hbm.at[ai.at[b]], rowsa[b], gsa[b]).start()
        pltpu.make_async_copy(feat_hbm.at[aj.at[b]], rowsb[b], gsb[b]).start()

    def grp(g, carry):
        for b in range(LBUF):
            k = g * LBUF + b
            pltpu.make_async_copy(feat_hbm.at[ai.at[k]], rowsa[b], gsa[b]).wait()
            pltpu.make_async_copy(feat_hbm.at[aj.at[k]], rowsb[b], gsb[b]).wait()

            def edge(e, c2):
                p = rowsa[b][e, pl.ds(0, 16)] * rowsb[b][e, pl.ds(0, 16)]
                p = p + rowsa[b][e, pl.ds(16, 16)] * rowsb[b][e, pl.ds(16, 16)]
                p = p + rowsa[b][e, pl.ds(32, 16)] * rowsb[b][e, pl.ds(32, 16)]
                p = p + rowsa[b][e, pl.ds(48, 16)] * rowsb[b][e, pl.ds(48, 16)]
                sbuf[e, :] = p
                return c2

            lax.fori_loop(0, CHUNK, edge, 0)

            @pl.when(k + LBUF < q)
            def _():
                pltpu.make_async_copy(
                    feat_hbm.at[ai.at[k + LBUF]], rowsa[b], gsa[b]).start()
                pltpu.make_async_copy(
                    feat_hbm.at[aj.at[k + LBUF]], rowsb[b], gsb[b]).start()

            pltpu.sync_copy(sbuf, out_hbm.at[pl.ds((start + k) * CHUNK, CHUNK)])
        return carry

    lax.fori_loop(0, q // LBUF, grp, 0)


_link_call = pl.kernel(
    _link_body,
    out_type=jax.ShapeDtypeStruct((LPAD, 16), jnp.float32),
    mesh=_mesh,
    compiler_params=pltpu.CompilerParams(use_tc_tiling_on_sc=False),
    scratch_types=[
        pltpu.VMEM((QM_L, CHUNK), jnp.int32),
        pltpu.VMEM((QM_L, CHUNK), jnp.int32),
        pltpu.VMEM((CHUNK, H), jnp.float32),
        pltpu.VMEM((CHUNK, H), jnp.float32),
        pltpu.VMEM((CHUNK, H), jnp.float32),
        pltpu.VMEM((CHUNK, H), jnp.float32),
        pltpu.VMEM((CHUNK, 16), jnp.float32),
        pltpu.SemaphoreType.DMA,
        pltpu.SemaphoreType.DMA,
        pltpu.SemaphoreType.DMA,
        pltpu.SemaphoreType.DMA,
    ],
)


def _tc_layer1(degp_ref, x_ref, g_ref, w1_ref, dinv_ref, y1_ref):
    deg = jnp.sum(degp_ref[...], axis=1, keepdims=True) + 1.0
    dinv = lax.rsqrt(jnp.maximum(deg, 1.0))
    xg = jnp.dot(x_ref[...], g_ref[...], preferred_element_type=jnp.float32)
    xw = jnp.dot(xg, w1_ref[...], preferred_element_type=jnp.float32)
    dinv_ref[...] = dinv
    y1_ref[...] = xw * dinv


def _tc_mid(accp_ref, y_ref, dinv_ref, b_ref, w_ref, out_ref, *, use_relu):
    dinv = dinv_ref[...]
    tot = accp_ref[0] + accp_ref[1] + y_ref[...]
    h = dinv * tot + b_ref[...][None, :]
    if use_relu:
        h = jnp.maximum(h, 0.0)
    out_ref[...] = jnp.dot(h, w_ref[...], preferred_element_type=jnp.float32) * dinv


def _tc_feat(accp_ref, y_ref, dinv_ref, b_ref, feat_ref, y4_ref):
    dinv = dinv_ref[...]
    tot = accp_ref[0] + accp_ref[1] + y_ref[...]
    feat = dinv * tot + b_ref[...][None, :]
    feat_ref[...] = feat
    y4_ref[...] = feat * dinv


def _tc_final(accp_ref, y4_ref, dinv_ref, wr_ref, br_ref, wk_ref, bk_ref,
              s2_ref, g16_ref, attr_ref, att_ref, res_ref):
    dinv = dinv_ref[...]
    p = dinv * (accp_ref[0] + accp_ref[1] + y4_ref[...])
    attr_l = jnp.dot(p, wr_ref[...], preferred_element_type=jnp.float32) + br_ref[...][None, :]
    att = jnp.dot(p, wk_ref[...], preferred_element_type=jnp.float32) + bk_ref[...][None, :]
    m = jnp.max(attr_l, axis=1, keepdims=True)
    lse = m + jnp.log(jnp.sum(jnp.exp(attr_l - m), axis=1, keepdims=True))
    attr_ref[...] = attr_l - lse
    att_ref[...] = att
    res_ref[...] = jnp.dot(s2_ref[...], g16_ref[...], preferred_element_type=jnp.float32)


def kernel(input, glove_matrix, W1, b1, W2, b2, W3, b3, Wattr, battr,
           Wattk, battk, edge_index, pos_edge_index, neg_edge_index):
    f32 = jnp.float32
    pad_e = jnp.full((EPAD - E,), N, jnp.int32)
    srcs2 = jnp.concatenate([edge_index[0], pad_e]).reshape(NCH_E, CHUNK)
    dsts2 = jnp.concatenate([edge_index[1], pad_e]).reshape(NCH_E, CHUNK)
    pad_l = jnp.full((LPAD - 2 * EP,), N, jnp.int32)
    li2 = jnp.concatenate([pos_edge_index[0], neg_edge_index[0], pad_l]).reshape(NCH_L, CHUNK)
    lj2 = jnp.concatenate([pos_edge_index[1], neg_edge_index[1], pad_l]).reshape(NCH_L, CHUNK)
    xpad = jnp.pad(input, ((0, NPAD - N), (0, 0)))
    zeros2 = jnp.zeros((NPAD, H), f32)
    g16 = (jnp.arange(256)[:, None] // 16 == jnp.arange(16)[None, :]).astype(f32)

    degp = jnp.transpose(_deg_call(dsts2))

    dinv, y1 = pl.pallas_call(
        _tc_layer1,
        out_shape=(jax.ShapeDtypeStruct((NPAD, 1), f32),
                   jax.ShapeDtypeStruct((NPAD, H), f32)),
    )(degp, xpad, glove_matrix, W1)

    acc1 = _prop_call(srcs2, dsts2, y1, zeros2)
    y2 = pl.pallas_call(
        functools.partial(_tc_mid, use_relu=True),
        out_shape=jax.ShapeDtypeStruct((NPAD, H), f32),
    )(acc1, y1, dinv, b1, W2)

    acc2 = _prop_call(srcs2, dsts2, y2, zeros2)
    y3 = pl.pallas_call(
        functools.partial(_tc_mid, use_relu=False),
        out_shape=jax.ShapeDtypeStruct((NPAD, H), f32),
    )(acc2, y2, dinv, b2, W3)

    acc3 = _prop_call(srcs2, dsts2, y3, zeros2)
    feat_pad, y4 = pl.pallas_call(
        _tc_feat,
        out_shape=(jax.ShapeDtypeStruct((NPAD, H), f32),
                   jax.ShapeDtypeStruct((NPAD, H), f32)),
    )(acc3, y3, dinv, b3)

    acc4 = _prop_call(srcs2, dsts2, y4, zeros2)
    smat = _link_call(li2, lj2, feat_pad)
    s2 = smat.reshape(LPAD * 16 // 256, 256)

    attr_ls, att, res2 = pl.pallas_call(
        _tc_final,
        out_shape=(jax.ShapeDtypeStruct((NPAD, C), f32),
                   jax.ShapeDtypeStruct((NPAD, C), f32),
                   jax.ShapeDtypeStruct((LPAD * 16 // 256, 16), f32)),
        compiler_params=pltpu.CompilerParams(vmem_limit_bytes=100 * 1024 * 1024),
    )(acc4, y4, dinv, Wattr, battr, Wattk, battk, s2, g16)

    res = res2.reshape(LPAD)[:2 * EP]
    return (res, attr_ls[:N], att[:N], feat_pad[:N])


# speedup vs baseline: 1.0054x; 1.0054x over previous
import functools

import jax
import jax.numpy as jnp
from jax import lax
from jax.experimental import pallas as pl
from jax.experimental.pallas import tpu as pltpu
from jax.experimental.pallas import tpu_sc as plsc

N = 10000
NPAD = 10112
F_IN = 128
H = 64
C = 40
E = 320000
EP = 100000

NC = 2
NS = 16
NT = NC * NS
RPT = NPAD // NS

CHUNK = 128
NBUF = 4
LBUF = 2

NCH_E = 2560
EPAD = NCH_E * CHUNK
Q0_E = 148
Q1_E = NCH_E // NS - Q0_E
QM_E = max(Q0_E, Q1_E)

NCH_L = 1600
LPAD = NCH_L * CHUNK
Q0_L = 92
Q1_L = NCH_L // NS - Q0_L
QM_L = max(Q0_L, Q1_L)

_mesh = plsc.VectorSubcoreMesh(
    core_axis_name="c", subcore_axis_name="s", num_cores=NC, num_subcores=NS)


def _tile_ids():
    c = lax.axis_index("c")
    s = lax.axis_index("s")
    return c, s, c * NS + s


def _split(c, s, q0, q1):
    q = jnp.where(c == 0, q0, q1)
    start = jnp.where(c == 0, s * q0, NS * q0 + s * q1)
    return q, start


def _deg_body(dsts_hbm, out_hbm, didx, deg):
    c, s, t = _tile_ids()
    cpt = NCH_E // NT
    pltpu.sync_copy(dsts_hbm.at[pl.ds(t * cpt, cpt)], didx)

    def z(i, carry):
        deg[pl.ds(i * 16, 16)] = jnp.zeros((16,), jnp.float32)
        return carry

    lax.fori_loop(0, NPAD // 16, z, 0)

    def g(i, carry):
        iv = didx[i // (CHUNK // 16), pl.ds((i % (CHUNK // 16)) * 16, 16)]
        plsc.addupdate_scatter(deg, [iv], jnp.ones((16,), jnp.float32))
        return carry

    lax.fori_loop(0, cpt * (CHUNK // 16), g, 0)
    pltpu.sync_copy(deg, out_hbm.at[t])


_deg_call = pl.kernel(
    _deg_body,
    out_type=jax.ShapeDtypeStruct((NT, NPAD), jnp.float32),
    mesh=_mesh,
    compiler_params=pltpu.CompilerParams(
        use_tc_tiling_on_sc=False, needs_layout_passes=False),
    scratch_types=[
        pltpu.VMEM((NCH_E // NT, CHUNK), jnp.int32),
        pltpu.VMEM((NPAD,), jnp.float32),
    ],
)


def _prop_body(srcs_hbm, dsts_hbm, y_hbm, zeros_hbm, out_hbm,
               sidx, didx, r0, r1, r2, r3, acc, g0, g1, g2, g3):
    c, s, t = _tile_ids()
    rows = [r0, r1, r2, r3]
    gsem = [g0, g1, g2, g3]
    q, start = _split(c, s, Q0_E, Q1_E)
    pltpu.sync_copy(srcs_hbm.at[pl.ds(start, QM_E)], sidx)
    pltpu.sync_copy(dsts_hbm.at[pl.ds(start, QM_E)], didx)
    pltpu.sync_copy(zeros_hbm.at[pl.ds(s * RPT, RPT)], acc.at[pl.ds(s * RPT, RPT)])
    plsc.subcore_barrier()
    for b in range(NBUF):
        pltpu.make_async_copy(y_hbm.at[sidx.at[b]], rows[b], gsem[b]).start()

    def grp(g, carry):
        for b in range(NBUF):
            k = g * NBUF + b
            pltpu.make_async_copy(y_hbm.at[sidx.at[k]], rows[b], gsem[b]).wait()
            pltpu.sync_copy(rows[b], acc.at[didx.at[k]], add=True)

            @pl.when(k + NBUF < q)
            def _():
                pltpu.make_async_copy(
                    y_hbm.at[sidx.at[k + NBUF]], rows[b], gsem[b]).start()
        return carry

    lax.fori_loop(0, q // NBUF, grp, 0)
    plsc.subcore_barrier()
    pltpu.sync_copy(acc.at[pl.ds(s * RPT, RPT)], out_hbm.at[c, pl.ds(s * RPT, RPT)])


_prop_call = pl.kernel(
    _prop_body,
    out_type=jax.ShapeDtypeStruct((NC, NPAD, H), jnp.float32),
    mesh=_mesh,
    compiler_params=pltpu.CompilerParams(use_tc_tiling_on_sc=False),
    scratch_types=[
        pltpu.VMEM((QM_E, CHUNK), jnp.int32),
        pltpu.VMEM((QM_E, CHUNK), jnp.int32),
        pltpu.VMEM((CHUNK, H), jnp.float32),
        pltpu.VMEM((CHUNK, H), jnp.float32),
        pltpu.VMEM((CHUNK, H), jnp.float32),
        pltpu.VMEM((CHUNK, H), jnp.float32),
        pltpu.VMEM_SHARED((NPAD, H), jnp.float32),
        pltpu.SemaphoreType.DMA,
        pltpu.SemaphoreType.DMA,
        pltpu.SemaphoreType.DMA,
        pltpu.SemaphoreType.DMA,
    ],
)


def _link_body(li_hbm, lj_hbm, feat_hbm, out_hbm, ai, aj,
               ra0, ra1, rb0, rb1, sbuf, ga0, ga1, gb0, gb1):
    c, s, t = _tile_ids()
    rowsa = [ra0, ra1]
    rowsb = [rb0, rb1]
    gsa = [ga0, ga1]
    gsb = [gb0, gb1]
    q, start = _split(c, s, Q0_L, Q1_L)
    pltpu.sync_copy(li_hbm.at[pl.ds(start, QM_L)], ai)
    pltpu.sync_copy(lj_hbm.at[pl.ds(start, QM_L)], aj)
    for b in range(LBUF):
        pltpu.make_async_copy(feat_hbm.at[ai.at[b]], rowsa[b], gsa[b]).start()
        pltpu.make_async_copy(feat_hbm.at[aj.at[b]], rowsb[b], gsb[b]).start()

    def grp(g, carry):
        for b in range(LBUF):
            k = g * LBUF + b
            pltpu.make_async_copy(feat_hbm.at[ai.at[k]], rowsa[b], gsa[b]).wait()
            pltpu.make_async_copy(feat_hbm.at[aj.at[k]], rowsb[b], gsb[b]).wait()

            def edge(e, c2):
                p = rowsa[b][e, pl.ds(0, 16)] * rowsb[b][e, pl.ds(0, 16)]
                p = p + rowsa[b][e, pl.ds(16, 16)] * rowsb[b][e, pl.ds(16, 16)]
                p = p + rowsa[b][e, pl.ds(32, 16)] * rowsb[b][e, pl.ds(32, 16)]
                p = p + rowsa[b][e, pl.ds(48, 16)] * rowsb[b][e, pl.ds(48, 16)]
                sbuf[e, :] = p
                return c2

            lax.fori_loop(0, CHUNK, edge, 0)

            @pl.when(k + LBUF < q)
            def _():
                pltpu.make_async_copy(
                    feat_hbm.at[ai.at[k + LBUF]], rowsa[b], gsa[b]).start()
                pltpu.make_async_copy(
                    feat_hbm.at[aj.at[k + LBUF]], rowsb[b], gsb[b]).start()

            pltpu.sync_copy(sbuf, out_hbm.at[pl.ds((start + k) * CHUNK, CHUNK)])
        return carry

    lax.fori_loop(0, q // LBUF, grp, 0)


_link_call = pl.kernel(
    _link_body,
    out_type=jax.ShapeDtypeStruct((LPAD, 16), jnp.float32),
    mesh=_mesh,
    compiler_params=pltpu.CompilerParams(use_tc_tiling_on_sc=False),
    scratch_types=[
        pltpu.VMEM((QM_L, CHUNK), jnp.int32),
        pltpu.VMEM((QM_L, CHUNK), jnp.int32),
        pltpu.VMEM((CHUNK, H), jnp.float32),
        pltpu.VMEM((CHUNK, H), jnp.float32),
        pltpu.VMEM((CHUNK, H), jnp.float32),
        pltpu.VMEM((CHUNK, H), jnp.float32),
        pltpu.VMEM((CHUNK, 16), jnp.float32),
        pltpu.SemaphoreType.DMA,
        pltpu.SemaphoreType.DMA,
        pltpu.SemaphoreType.DMA,
        pltpu.SemaphoreType.DMA,
    ],
)


def _tc_layer1(degp_ref, x_ref, g_ref, w1_ref, dinv_ref, y1_ref):
    deg = jnp.sum(degp_ref[...], axis=1, keepdims=True) + 1.0
    dinv = lax.rsqrt(jnp.maximum(deg, 1.0))
    xg = jnp.dot(x_ref[...], g_ref[...], preferred_element_type=jnp.float32)
    xw = jnp.dot(xg, w1_ref[...], preferred_element_type=jnp.float32)
    dinv_ref[...] = dinv
    y1_ref[...] = xw * dinv


def _tc_mid(accp_ref, y_ref, dinv_ref, b_ref, w_ref, out_ref, *, use_relu):
    dinv = dinv_ref[...]
    tot = accp_ref[0] + accp_ref[1] + y_ref[...]
    h = dinv * tot + b_ref[...][None, :]
    if use_relu:
        h = jnp.maximum(h, 0.0)
    out_ref[...] = jnp.dot(h, w_ref[...], preferred_element_type=jnp.float32) * dinv


def _tc_feat(accp_ref, y_ref, dinv_ref, b_ref, feat_ref, y4_ref):
    dinv = dinv_ref[...]
    tot = accp_ref[0] + accp_ref[1] + y_ref[...]
    feat = dinv * tot + b_ref[...][None, :]
    feat_ref[...] = feat
    y4_ref[...] = feat * dinv


def _tc_final(accp_ref, y4_ref, dinv_ref, wr_ref, br_ref, wk_ref, bk_ref,
              s2_ref, g16_ref, attr_ref, att_ref, res_ref):
    dinv = dinv_ref[...]
    p = dinv * (accp_ref[0] + accp_ref[1] + y4_ref[...])
    attr_l = jnp.dot(p, wr_ref[...], preferred_element_type=jnp.float32) + br_ref[...][None, :]
    att = jnp.dot(p, wk_ref[...], preferred_element_type=jnp.float32) + bk_ref[...][None, :]
    m = jnp.max(attr_l, axis=1, keepdims=True)
    lse = m + jnp.log(jnp.sum(jnp.exp(attr_l - m), axis=1, keepdims=True))
    attr_ref[...] = attr_l - lse
    att_ref[...] = att
    res_ref[...] = jnp.dot(s2_ref[...], g16_ref[...], preferred_element_type=jnp.float32)


def kernel(input, glove_matrix, W1, b1, W2, b2, W3, b3, Wattr, battr,
           Wattk, battk, edge_index, pos_edge_index, neg_edge_index):
    f32 = jnp.float32
    pad_e = jnp.full((EPAD - E,), N, jnp.int32)
    srcs2 = jnp.concatenate([edge_index[0], pad_e]).reshape(NCH_E, CHUNK)
    dsts2 = jnp.concatenate([edge_index[1], pad_e]).reshape(NCH_E, CHUNK)
    pad_l = jnp.full((LPAD - 2 * EP,), N, jnp.int32)
    li2 = jnp.concatenate([pos_edge_index[0], neg_edge_index[0], pad_l]).reshape(NCH_L, CHUNK)
    lj2 = jnp.concatenate([pos_edge_index[1], neg_edge_index[1], pad_l]).reshape(NCH_L, CHUNK)
    xpad = jnp.pad(input, ((0, NPAD - N), (0, 0)))
    zeros2 = jnp.zeros((NPAD, H), f32)
    g16 = (jnp.arange(256)[:, None] // 16 == jnp.arange(16)[None, :]).astype(f32)

    degp = jnp.transpose(_deg_call(dsts2))

    dinv, y1 = pl.pallas_call(
        _tc_layer1,
        out_shape=(jax.ShapeDtypeStruct((NPAD, 1), f32),
                   jax.ShapeDtypeStruct((NPAD, H), f32)),
    )(degp, xpad, glove_matrix, W1)

    acc1 = _prop_call(srcs2, dsts2, y1, zeros2)
    y2 = pl.pallas_call(
        functools.partial(_tc_mid, use_relu=True),
        out_shape=jax.ShapeDtypeStruct((NPAD, H), f32),
    )(acc1, y1, dinv, b1, W2)

    acc2 = _prop_call(srcs2, dsts2, y2, zeros2)
    y3 = pl.pallas_call(
        functools.partial(_tc_mid, use_relu=False),
        out_shape=jax.ShapeDtypeStruct((NPAD, H), f32),
    )(acc2, y2, dinv, b2, W3)

    acc3 = _prop_call(srcs2, dsts2, y3, zeros2)
    feat_pad, y4 = pl.pallas_call(
        _tc_feat,
        out_shape=(jax.ShapeDtypeStruct((NPAD, H), f32),
                   jax.ShapeDtypeStruct((NPAD, H), f32)),
    )(acc3, y3, dinv, b3)

    acc4 = _prop_call(srcs2, dsts2, y4, zeros2)
    smat = _link_call(li2, lj2, feat_pad)
    s2 = smat.reshape(LPAD * 16 // 256, 256)

    attr_ls, att, res2 = pl.pallas_call(
        _tc_final,
        out_shape=(jax.ShapeDtypeStruct((NPAD, C), f32),
                   jax.ShapeDtypeStruct((NPAD, C), f32),
                   jax.ShapeDtypeStruct((LPAD * 16 // 256, 16), f32)),
        compiler_params=pltpu.CompilerParams(vmem_limit_bytes=100 * 1024 * 1024),
    )(acc4, y4, dinv, Wattr, battr, Wattk, battk, s2, g16)

    res = res2.reshape(LPAD)[:2 * EP]
    return (res, attr_ls[:N], att[:N], feat_pad[:N])
